# speedup vs baseline: 1.1076x; 1.1076x over previous
import jax
import jax.numpy as jnp
from jax.experimental import pallas as pl
from jax.experimental.pallas import tpu as pltpu
from jax.experimental.pallas import tpu_sc as plsc

NE = 8192
ED = 256
NT = 16384
BM = 256
GW = 128


def _dist_argmin_kernel(x_ref, w2_ref, xsq_ref, wsq_ref, idx_ref):
    mm = jax.lax.dot_general(
        x_ref[...], w2_ref[...],
        dimension_numbers=(((1,), (1,)), ((), ())),
        preferred_element_type=jnp.float32,
        precision=jax.lax.Precision.DEFAULT)
    s = (xsq_ref[...] + wsq_ref[...]) - mm
    rowmin = jnp.min(s, axis=1, keepdims=True)
    iota = jax.lax.broadcasted_iota(jnp.int32, s.shape, 1)
    cand = jnp.where(s == rowmin, iota, jnp.int32(NE))
    idx_ref[...] = jnp.min(cand, axis=1, keepdims=True)


def _sc_gather(W, idx):
    mesh = plsc.VectorSubcoreMesh(core_axis_name="c", subcore_axis_name="s")
    idx2 = idx.reshape(1, NT)

    @pl.kernel(out_type=jax.ShapeDtypeStruct((NT, ED), jnp.float32),
               mesh=mesh)
    def k(w_hbm, i_hbm, o_hbm):
        def body(i_vmem, o_vmem):
            pltpu.sync_copy(w_hbm.at[i_vmem.at[0]], o_vmem)

        pltpu.emit_pipeline(
            body,
            grid=(NT // GW,),
            in_specs=[pl.BlockSpec((1, GW), index_map=lambda i: (0, i))],
            out_specs=[pl.BlockSpec((GW, ED), index_map=lambda i: (i, 0))],
            core_axis_name=("c", "s"),
            dimension_semantics=(pltpu.PARALLEL,),
        )(i_hbm, o_hbm)

    return k(W, idx2)


def kernel(x, W):
    xsq = jnp.sum(x ** 2, axis=1, keepdims=True)
    wsq = jnp.sum(W ** 2, axis=1)[None, :]
    w2 = 2.0 * W

    idx2d = pl.pallas_call(
        _dist_argmin_kernel,
        grid=(NT // BM,),
        in_specs=[
            pl.BlockSpec((BM, ED), lambda i: (i, 0)),
            pl.BlockSpec((NE, ED), lambda i: (0, 0)),
            pl.BlockSpec((BM, 1), lambda i: (i, 0)),
            pl.BlockSpec((1, NE), lambda i: (0, 0)),
        ],
        out_specs=pl.BlockSpec((BM, 1), lambda i: (i, 0)),
        out_shape=jax.ShapeDtypeStruct((NT, 1), jnp.int32),
    )(x, w2, xsq, wsq)

    min_indices = idx2d[:, 0]
    z_q = _sc_gather(W, min_indices)
    return (z_q, min_indices)


# speedup vs baseline: 1.1989x; 1.0825x over previous
import jax
import jax.numpy as jnp
from jax.experimental import pallas as pl
from jax.experimental.pallas import tpu as pltpu
from jax.experimental.pallas import tpu_sc as plsc

NE = 8192
ED = 256
NT = 16384
BM = 256
GW = 128


def _dist_argmin_kernel(x_ref, w2_ref, xsq_ref, wsq_ref, idx_ref):
    mm = jax.lax.dot_general(
        x_ref[...], w2_ref[...],
        dimension_numbers=(((1,), (1,)), ((), ())),
        preferred_element_type=jnp.float32,
        precision=jax.lax.Precision.DEFAULT)
    s = (xsq_ref[...] + wsq_ref[...]) - mm
    rowmin = jnp.min(s, axis=1, keepdims=True)
    iota = jax.lax.broadcasted_iota(jnp.int32, s.shape, 1).astype(jnp.float32)
    cand = jnp.where(s == rowmin, iota, jnp.float32(NE))
    idx_ref[...] = jnp.min(cand, axis=1, keepdims=True).astype(jnp.int32)


def _sc_gather(W, idx):
    mesh = plsc.VectorSubcoreMesh(core_axis_name="c", subcore_axis_name="s")
    idx2 = idx.reshape(1, NT)

    @pl.kernel(out_type=jax.ShapeDtypeStruct((NT, ED), jnp.float32),
               mesh=mesh)
    def k(w_hbm, i_hbm, o_hbm):
        def body(i_vmem, o_vmem):
            pltpu.sync_copy(w_hbm.at[i_vmem.at[0]], o_vmem)

        pltpu.emit_pipeline(
            body,
            grid=(NT // GW,),
            in_specs=[pl.BlockSpec((1, GW), index_map=lambda i: (0, i))],
            out_specs=[pl.BlockSpec((GW, ED), index_map=lambda i: (i, 0))],
            core_axis_name=("c", "s"),
            dimension_semantics=(pltpu.PARALLEL,),
        )(i_hbm, o_hbm)

    return k(W, idx2)


def kernel(x, W):
    xsq = jnp.sum(x ** 2, axis=1, keepdims=True)
    wsq = jnp.sum(W ** 2, axis=1)[None, :]
    w2 = 2.0 * W

    idx2d = pl.pallas_call(
        _dist_argmin_kernel,
        grid=(NT // BM,),
        in_specs=[
            pl.BlockSpec((BM, ED), lambda i: (i, 0)),
            pl.BlockSpec((NE, ED), lambda i: (0, 0)),
            pl.BlockSpec((BM, 1), lambda i: (i, 0)),
            pl.BlockSpec((1, NE), lambda i: (0, 0)),
        ],
        out_specs=pl.BlockSpec((BM, 1), lambda i: (i, 0)),
        out_shape=jax.ShapeDtypeStruct((NT, 1), jnp.int32),
    )(x, w2, xsq, wsq)

    min_indices = idx2d[:, 0]
    z_q = _sc_gather(W, min_indices)
    return (z_q, min_indices)
